# baseline (speedup 1.0000x reference)
import functools

import jax
import jax.numpy as jnp
from jax import lax
from jax.experimental import pallas as pl
from jax.experimental.pallas import tpu as pltpu
from jax.experimental.pallas import tpu_sc as plsc

N = 10000
E = 320000
D = 128
H = 64
Bn = 1000
NBLK = N // Bn

CH = 88
NS = 16
CPS = 228
NCH = CPS * NS
EPAD = NCH * CH
GRP = 12
NGRP = CPS // GRP
ACC_R = 2 * N + 8
NZF = 227
ZT = ACC_R - NZF * CH
WCH = 1000
NWCH = (2 * N) // WCH


def _leaky(v):
    return jnp.where(v >= 0, v, 0.01 * v)


def _table_write(x, wr_ref, t_ref):
    xr0 = jnp.dot(x, wr_ref[0])
    xr1 = jnp.dot(x, wr_ref[1])
    t_ref[0] = xr0[:, :H]
    t_ref[1] = xr1[:, :H]
    t_ref[2] = xr0[:, H:]
    t_ref[3] = xr1[:, H:]


def _k1_body(cat_ref, wc_ref, bc_ref, wi_ref, bi_ref, wr_ref, x_ref, t_ref):
    c = _leaky(jnp.dot(cat_ref[...], wc_ref[...]) + bc_ref[...])
    x = _leaky(jnp.dot(c, wi_ref[...]) + bi_ref[...])
    x_ref[...] = x
    _table_write(x, wr_ref, t_ref)


def _combine(x, wroot_ref, br_ref, a00, a01, a10, a11, c0, c1):
    cnt0 = jnp.maximum(c0[0][:, 0:1], 1.0)
    cnt1 = jnp.maximum(c1[0][:, 0:1], 1.0)
    lo = a00[0] / cnt0 + a01[0] / cnt1
    hi = a10[0] / cnt0 + a11[0] / cnt1
    return (jnp.dot(x, wroot_ref[...]) + br_ref[...]
            + jnp.concatenate([lo, hi], axis=1))


def _k2_body(x_ref, wroot_ref, br_ref, a00, a01, a10, a11, c0, c1, wr_ref,
             x2_ref, t_ref):
    x2 = _combine(x_ref[...], wroot_ref, br_ref, a00, a01, a10, a11, c0, c1)
    x2_ref[...] = x2
    _table_write(x2, wr_ref, t_ref)


def _k3_body(x_ref, wroot_ref, br_ref, a00, a01, a10, a11, c0, c1,
             wo1_ref, bo1_ref, wo2_ref, bo2_ref, out_ref):
    x3 = _combine(x_ref[...], wroot_ref, br_ref, a00, a01, a10, a11, c0, c1)
    x4 = _leaky(jnp.dot(x3, wo1_ref[...]) + bo1_ref[...])
    out_ref[...] = jnp.dot(x4, wo2_ref[...]) + bo2_ref[...]


def _full(shape):
    return pl.BlockSpec(shape, lambda i: (0,) * len(shape))


def _agg_specs():
    return [
        pl.BlockSpec((1, Bn, H), lambda i: (0, i, 0)),
        pl.BlockSpec((1, Bn, H), lambda i: (0, NBLK + i, 0)),
        pl.BlockSpec((1, Bn, H), lambda i: (1, i, 0)),
        pl.BlockSpec((1, Bn, H), lambda i: (1, NBLK + i, 0)),
    ]


def _cnt_specs():
    return [
        pl.BlockSpec((1, Bn, 16), lambda i: (0, i, 0)),
        pl.BlockSpec((1, Bn, 16), lambda i: (0, NBLK + i, 0)),
    ]


_k1 = pl.pallas_call(
    _k1_body,
    grid=(NBLK,),
    in_specs=[
        pl.BlockSpec((Bn, 11), lambda i: (i, 0)),
        _full((11, D)), _full((1, D)), _full((D, D)), _full((1, D)),
        _full((2, D, D)),
    ],
    out_specs=[
        pl.BlockSpec((Bn, D), lambda i: (i, 0)),
        pl.BlockSpec((4, Bn, H), lambda i: (0, i, 0)),
    ],
    out_shape=[
        jax.ShapeDtypeStruct((N, D), jnp.float32),
        jax.ShapeDtypeStruct((4, N, H), jnp.float32),
    ],
)

_k2 = pl.pallas_call(
    _k2_body,
    grid=(NBLK,),
    in_specs=[
        pl.BlockSpec((Bn, D), lambda i: (i, 0)),
        _full((D, D)), _full((1, D)),
        *_agg_specs(), *_cnt_specs(),
        _full((2, D, D)),
    ],
    out_specs=[
        pl.BlockSpec((Bn, D), lambda i: (i, 0)),
        pl.BlockSpec((4, Bn, H), lambda i: (0, i, 0)),
    ],
    out_shape=[
        jax.ShapeDtypeStruct((N, D), jnp.float32),
        jax.ShapeDtypeStruct((4, N, H), jnp.float32),
    ],
)

_k3 = pl.pallas_call(
    _k3_body,
    grid=(NBLK,),
    in_specs=[
        pl.BlockSpec((Bn, D), lambda i: (i, 0)),
        _full((D, D)), _full((1, D)),
        *_agg_specs(), *_cnt_specs(),
        _full((D, D)), _full((1, D)), _full((D, 2)), _full((1, 2)),
    ],
    out_specs=pl.BlockSpec((Bn, 2), lambda i: (i, 0)),
    out_shape=jax.ShapeDtypeStruct((N, 2), jnp.float32),
)


def _sc_body(with_cnt, nb, a, tab, gx, sx, *rest):
    if with_cnt:
        (agg_out, cnt_out, acc, cntacc, ones, zb16, gbuf, sbuf,
         rows, frows) = rest[:10]
        gsem = list(rest[10:10 + nb])
        ssem = list(rest[10 + nb:10 + nb + 2])
        cn = rest[10 + nb + 2]
    else:
        (agg_out, acc, gbuf, sbuf, rows, frows) = rest[:6]
        gsem = list(rest[6:6 + nb])
        ssem = list(rest[6 + nb:6 + nb + 2])
        cnt_out = cntacc = ones = zb16 = cn = None
    c = lax.axis_index("c")
    s = lax.axis_index("s")

    def _zr(i, carry):
        def _zc(j, carry2):
            for p in range(2):
                frows[p, i, pl.ds(j * 16, 16)] = jnp.zeros((16,),
                                                           jnp.float32)
            return carry2
        return lax.fori_loop(0, H // 16, _zc, carry)
    lax.fori_loop(0, CH, _zr, 0)

    def _zs(i, carry):
        sbuf[0, pl.ds(i * 16, 16)] = jnp.zeros((16,), jnp.int32)
        return carry
    lax.fori_loop(0, CH // 16, _zs, 0)
    sbuf[0, pl.ds(CH - 16, 16)] = jnp.zeros((16,), jnp.int32)

    if with_cnt:
        def _zo(i, carry):
            zb16[i, pl.ds(0, 16)] = jnp.zeros((16,), jnp.float32)
            ones[i, pl.ds(0, 16)] = jnp.ones((16,), jnp.float32)
            return carry
        lax.fori_loop(0, CH, _zo, 0)

    def _za(j, carry):
        k = s + NS * j

        @pl.when(k < NZF)
        def _():
            pltpu.sync_copy(frows.at[0], acc.at[pl.ds(k * CH, CH)])
            if with_cnt:
                pltpu.sync_copy(zb16, cntacc.at[pl.ds(k * CH, CH)])

        @pl.when(k == NZF)
        def _():
            pltpu.sync_copy(frows.at[0, pl.ds(0, ZT)],
                            acc.at[pl.ds(NZF * CH, ZT)])
            if with_cnt:
                pltpu.sync_copy(zb16.at[pl.ds(0, ZT)],
                                cntacc.at[pl.ds(NZF * CH, ZT)])
        return carry
    lax.fori_loop(0, (NZF + NS) // NS + 1, _za, 0)

    plsc.subcore_barrier()

    for p in range(2):
        pltpu.async_copy(frows.at[p], acc.at[sbuf.at[0]], ssem[p], add=True)

    def _outer(it, carry):
        base = s * CPS + it * GRP
        pltpu.sync_copy(gx.at[c, pl.ds(base, GRP)], gbuf)
        pltpu.sync_copy(sx.at[pl.ds(base, GRP)], sbuf)
        gd = [None] * GRP
        for k in range(a):
            gd[k] = pltpu.async_copy(tab.at[gbuf.at[k]], rows.at[k % nb],
                                     gsem[k % nb])
        for k in range(GRP):
            b = k % nb
            p = k % 2
            if k + a < GRP:
                gd[k + a] = pltpu.async_copy(tab.at[gbuf.at[k + a]],
                                             rows.at[(k + a) % nb],
                                             gsem[(k + a) % nb])
            gd[k].wait()
            pltpu.make_async_copy(frows.at[p], acc.at[sbuf.at[0]],
                                  ssem[p]).wait()

            def _cv(i, carry2):
                w0 = rows[b, i, pl.ds(0, 16)]
                w1 = rows[b, i, pl.ds(16, 16)]
                sh16 = jnp.full((16,), 16, jnp.int32)
                hi_mask = jnp.full((16,), -65536, jnp.int32)
                a0 = lax.bitcast_convert_type(w0 << sh16, jnp.float32)
                b0 = lax.bitcast_convert_type(w0 & hi_mask, jnp.float32)
                a1 = lax.bitcast_convert_type(w1 << sh16, jnp.float32)
                b1 = lax.bitcast_convert_type(w1 & hi_mask, jnp.float32)
                frows[p, i, pl.ds(0, 16)] = a0
                frows[p, i, pl.ds(32, 16)] = b0
                frows[p, i, pl.ds(16, 16)] = a1
                frows[p, i, pl.ds(48, 16)] = b1
                return carry2
            lax.fori_loop(0, CH, _cv, 0)

            pltpu.async_copy(frows.at[p], acc.at[sbuf.at[k]], ssem[p],
                             add=True)
            if with_cnt:
                pltpu.async_copy(ones, cntacc.at[sbuf.at[k]], cn, add=True)
        return carry
    lax.fori_loop(0, NGRP, _outer, 0)

    for p in range(2):
        pltpu.make_async_copy(frows.at[p], acc.at[sbuf.at[0]],
                              ssem[p]).wait()
    if with_cnt:
        def _dr(i, carry):
            pltpu.make_async_copy(ones, cntacc.at[sbuf.at[0]], cn).wait()
            return carry
        lax.fori_loop(0, CPS, _dr, 0)

    plsc.subcore_barrier()

    def _wb(j, carry):
        k = s + NS * j

        @pl.when(k < NWCH)
        def _():
            pltpu.sync_copy(acc.at[pl.ds(k * WCH, WCH)],
                            agg_out.at[c, pl.ds(k * WCH, WCH)])
            if with_cnt:
                pltpu.sync_copy(cntacc.at[pl.ds(k * WCH, WCH)],
                                cnt_out.at[c, pl.ds(k * WCH, WCH)])
        return carry
    lax.fori_loop(0, (NWCH + NS - 1) // NS, _wb, 0)


def _make_sc(with_cnt, nb, a):
    out_type = [jax.ShapeDtypeStruct((2, 2 * N, H), jnp.float32)]
    scratch = [
        pltpu.VMEM_SHARED((ACC_R, H), jnp.float32),
    ]
    if with_cnt:
        out_type.append(jax.ShapeDtypeStruct((2, 2 * N, 16), jnp.float32))
        scratch += [
            pltpu.VMEM_SHARED((ACC_R, 16), jnp.float32),
            pltpu.VMEM((CH, 16), jnp.float32),
            pltpu.VMEM((CH, 16), jnp.float32),
        ]
    scratch += [
        pltpu.VMEM((GRP, CH), jnp.int32),
        pltpu.VMEM((GRP, CH), jnp.int32),
        pltpu.VMEM((nb, CH, H // 2), jnp.int32),
        pltpu.VMEM((2, CH, H), jnp.float32),
    ]
    scratch += [pltpu.SemaphoreType.DMA] * (nb + 2)
    if with_cnt:
        scratch.append(pltpu.SemaphoreType.DMA)
    return pl.kernel(
        functools.partial(_sc_body, with_cnt, nb, a),
        out_type=tuple(out_type) if with_cnt else out_type[0],
        mesh=plsc.VectorSubcoreMesh(core_axis_name="c", subcore_axis_name="s"),
        scratch_types=scratch,
        compiler_params=pltpu.CompilerParams(use_tc_tiling_on_sc=False),
    )


_sc1 = _make_sc(True, 3, 2)
_sc2 = _make_sc(False, 6, 4)


def kernel(des, tweet, num_prop, cat_prop, edge_index, edge_type,
           W_cat, b_cat, W_in, b_in, W_rel, W_root, b_rgcn,
           W_o1, b_o1, W_o2, b_o2):
    src = edge_index[0].astype(jnp.int32)
    dst = edge_index[1].astype(jnp.int32)
    et = edge_type.astype(jnp.int32)
    g0 = jnp.pad(et * N + src, (0, EPAD - E))
    gx = jnp.stack([g0, g0 + 2 * N]).reshape(2, NCH, CH)
    sx = jnp.pad(et * N + dst, (0, EPAD - E),
                 constant_values=2 * N).reshape(NCH, CH)

    bc = b_cat.reshape(1, D)
    bi = b_in.reshape(1, D)
    br = b_rgcn.reshape(1, D)
    bo1 = b_o1.reshape(1, D)
    bo2 = b_o2.reshape(1, 2)

    def _shuf(t):
        tb = (t.reshape(4, N, 2, H // 2).transpose(0, 1, 3, 2)
              .reshape(4 * N, H // 2, 2).astype(jnp.bfloat16))
        return jax.lax.bitcast_convert_type(tb, jnp.int32)

    x1, t1 = _k1(cat_prop, W_cat, bc, W_in, bi, W_rel)
    agg1, cnt16 = _sc1(_shuf(t1), gx, sx)
    x2, t2 = _k2(x1, W_root, br, agg1, agg1, agg1, agg1, cnt16, cnt16, W_rel)
    agg2 = _sc2(_shuf(t2), gx, sx)
    return _k3(x2, W_root, br, agg2, agg2, agg2, agg2, cnt16, cnt16,
               W_o1, bo1, W_o2, bo2)


# speedup vs baseline: 1.0799x; 1.0799x over previous
import functools

import jax
import jax.numpy as jnp
from jax import lax
from jax.experimental import pallas as pl
from jax.experimental.pallas import tpu as pltpu
from jax.experimental.pallas import tpu_sc as plsc

N = 10000
E = 320000
D = 128
H = 64
Bn = 1000
NBLK = N // Bn

CH = 88
NS = 16
CPS = 228
NCH = CPS * NS
EPAD = NCH * CH
GRP = 12
NGRP = CPS // GRP
ACC_R = 2 * N + 8
NZF = 227
ZT = ACC_R - NZF * CH
WCH = 1000
NWCH = (2 * N) // WCH


def _leaky(v):
    return jnp.where(v >= 0, v, 0.01 * v)


def _table_write(x, wr_ref, t_ref):
    xr0 = jnp.dot(x, wr_ref[0])
    xr1 = jnp.dot(x, wr_ref[1])
    t_ref[0] = xr0[:, :H]
    t_ref[1] = xr1[:, :H]
    t_ref[2] = xr0[:, H:]
    t_ref[3] = xr1[:, H:]


def _k1_body(cat_ref, wc_ref, bc_ref, wi_ref, bi_ref, wr_ref, x_ref, t_ref):
    c = _leaky(jnp.dot(cat_ref[...], wc_ref[...]) + bc_ref[...])
    x = _leaky(jnp.dot(c, wi_ref[...]) + bi_ref[...])
    x_ref[...] = x
    _table_write(x, wr_ref, t_ref)


def _combine(x, wroot_ref, br_ref, a00, a01, a10, a11, c0, c1):
    cnt0 = jnp.maximum(c0[0][:, 0:1], 1.0)
    cnt1 = jnp.maximum(c1[0][:, 0:1], 1.0)
    lo = a00[0] / cnt0 + a01[0] / cnt1
    hi = a10[0] / cnt0 + a11[0] / cnt1
    return (jnp.dot(x, wroot_ref[...]) + br_ref[...]
            + jnp.concatenate([lo, hi], axis=1))


def _k2_body(x_ref, wroot_ref, br_ref, a00, a01, a10, a11, c0, c1, wr_ref,
             x2_ref, t_ref):
    x2 = _combine(x_ref[...], wroot_ref, br_ref, a00, a01, a10, a11, c0, c1)
    x2_ref[...] = x2
    _table_write(x2, wr_ref, t_ref)


def _k3_body(x_ref, wroot_ref, br_ref, a00, a01, a10, a11, c0, c1,
             wo1_ref, bo1_ref, wo2_ref, bo2_ref, out_ref):
    x3 = _combine(x_ref[...], wroot_ref, br_ref, a00, a01, a10, a11, c0, c1)
    x4 = _leaky(jnp.dot(x3, wo1_ref[...]) + bo1_ref[...])
    out_ref[...] = jnp.dot(x4, wo2_ref[...]) + bo2_ref[...]


def _full(shape):
    return pl.BlockSpec(shape, lambda i: (0,) * len(shape))


def _agg_specs():
    return [
        pl.BlockSpec((1, Bn, H), lambda i: (0, i, 0)),
        pl.BlockSpec((1, Bn, H), lambda i: (0, NBLK + i, 0)),
        pl.BlockSpec((1, Bn, H), lambda i: (1, i, 0)),
        pl.BlockSpec((1, Bn, H), lambda i: (1, NBLK + i, 0)),
    ]


def _cnt_specs():
    return [
        pl.BlockSpec((1, Bn, 16), lambda i: (0, i, 0)),
        pl.BlockSpec((1, Bn, 16), lambda i: (0, NBLK + i, 0)),
    ]


_k1 = pl.pallas_call(
    _k1_body,
    grid=(NBLK,),
    in_specs=[
        pl.BlockSpec((Bn, 11), lambda i: (i, 0)),
        _full((11, D)), _full((1, D)), _full((D, D)), _full((1, D)),
        _full((2, D, D)),
    ],
    out_specs=[
        pl.BlockSpec((Bn, D), lambda i: (i, 0)),
        pl.BlockSpec((4, Bn, H), lambda i: (0, i, 0)),
    ],
    out_shape=[
        jax.ShapeDtypeStruct((N, D), jnp.float32),
        jax.ShapeDtypeStruct((4, N, H), jnp.float32),
    ],
)

_k2 = pl.pallas_call(
    _k2_body,
    grid=(NBLK,),
    in_specs=[
        pl.BlockSpec((Bn, D), lambda i: (i, 0)),
        _full((D, D)), _full((1, D)),
        *_agg_specs(), *_cnt_specs(),
        _full((2, D, D)),
    ],
    out_specs=[
        pl.BlockSpec((Bn, D), lambda i: (i, 0)),
        pl.BlockSpec((4, Bn, H), lambda i: (0, i, 0)),
    ],
    out_shape=[
        jax.ShapeDtypeStruct((N, D), jnp.float32),
        jax.ShapeDtypeStruct((4, N, H), jnp.float32),
    ],
)

_k3 = pl.pallas_call(
    _k3_body,
    grid=(NBLK,),
    in_specs=[
        pl.BlockSpec((Bn, D), lambda i: (i, 0)),
        _full((D, D)), _full((1, D)),
        *_agg_specs(), *_cnt_specs(),
        _full((D, D)), _full((1, D)), _full((D, 2)), _full((1, 2)),
    ],
    out_specs=pl.BlockSpec((Bn, 2), lambda i: (i, 0)),
    out_shape=jax.ShapeDtypeStruct((N, 2), jnp.float32),
)


def _sc_body(with_cnt, nb, a, tab, gx, sx, *rest):
    if with_cnt:
        (agg_out, cnt_out, acc, cntacc, ones, zb16, gbuf, sbuf,
         rows, frows) = rest[:10]
        gsem = list(rest[10:10 + nb])
        ssem = list(rest[10 + nb:10 + nb + 2])
        cn = rest[10 + nb + 2]
    else:
        (agg_out, acc, gbuf, sbuf, rows, frows) = rest[:6]
        gsem = list(rest[6:6 + nb])
        ssem = list(rest[6 + nb:6 + nb + 2])
        cnt_out = cntacc = ones = zb16 = cn = None
    c = lax.axis_index("c")
    s = lax.axis_index("s")

    def _zr(i, carry):
        def _zc(j, carry2):
            for p in range(2):
                frows[p, i, pl.ds(j * 16, 16)] = jnp.zeros((16,),
                                                           jnp.float32)
            return carry2
        return lax.fori_loop(0, H // 16, _zc, carry)
    lax.fori_loop(0, CH, _zr, 0)

    def _zs(i, carry):
        sbuf[0, pl.ds(i * 16, 16)] = jnp.zeros((16,), jnp.int32)
        return carry
    lax.fori_loop(0, CH // 16, _zs, 0)
    sbuf[0, pl.ds(CH - 16, 16)] = jnp.zeros((16,), jnp.int32)

    if with_cnt:
        def _zo(i, carry):
            zb16[i, pl.ds(0, 16)] = jnp.zeros((16,), jnp.float32)
            ones[i, pl.ds(0, 16)] = jnp.ones((16,), jnp.float32)
            return carry
        lax.fori_loop(0, CH, _zo, 0)

    def _za(j, carry):
        k = s + NS * j

        @pl.when(k < NZF)
        def _():
            pltpu.sync_copy(frows.at[0], acc.at[pl.ds(k * CH, CH)])
            if with_cnt:
                pltpu.sync_copy(zb16, cntacc.at[pl.ds(k * CH, CH)])

        @pl.when(k == NZF)
        def _():
            pltpu.sync_copy(frows.at[0, pl.ds(0, ZT)],
                            acc.at[pl.ds(NZF * CH, ZT)])
            if with_cnt:
                pltpu.sync_copy(zb16.at[pl.ds(0, ZT)],
                                cntacc.at[pl.ds(NZF * CH, ZT)])
        return carry
    lax.fori_loop(0, (NZF + NS) // NS + 1, _za, 0)

    plsc.subcore_barrier()

    for p in range(2):
        pltpu.async_copy(frows.at[p], acc.at[sbuf.at[0]], ssem[p], add=True)

    def _outer(it, carry):
        base = s * CPS + it * GRP
        pltpu.sync_copy(gx.at[c, pl.ds(base, GRP)], gbuf)
        pltpu.sync_copy(sx.at[pl.ds(base, GRP)], sbuf)
        gd = [None] * GRP
        for k in range(a):
            gd[k] = pltpu.async_copy(tab.at[gbuf.at[k]], rows.at[k % nb],
                                     gsem[k % nb])
        for k in range(GRP):
            b = k % nb
            p = k % 2
            if k + a < GRP:
                gd[k + a] = pltpu.async_copy(tab.at[gbuf.at[k + a]],
                                             rows.at[(k + a) % nb],
                                             gsem[(k + a) % nb])
            gd[k].wait()
            pltpu.make_async_copy(frows.at[p], acc.at[sbuf.at[0]],
                                  ssem[p]).wait()

            def _cv(i0, carry2):
                sh16 = jnp.full((16,), 16, jnp.int32)
                hi_mask = jnp.full((16,), -65536, jnp.int32)
                for r in range(8):
                    i = i0 * 8 + r
                    w0 = rows[b, i, pl.ds(0, 16)]
                    w1 = rows[b, i, pl.ds(16, 16)]
                    frows[p, i, pl.ds(0, 16)] = lax.bitcast_convert_type(
                        w0 << sh16, jnp.float32)
                    frows[p, i, pl.ds(32, 16)] = lax.bitcast_convert_type(
                        w0 & hi_mask, jnp.float32)
                    frows[p, i, pl.ds(16, 16)] = lax.bitcast_convert_type(
                        w1 << sh16, jnp.float32)
                    frows[p, i, pl.ds(48, 16)] = lax.bitcast_convert_type(
                        w1 & hi_mask, jnp.float32)
                return carry2
            lax.fori_loop(0, CH // 8, _cv, 0)

            pltpu.async_copy(frows.at[p], acc.at[sbuf.at[k]], ssem[p],
                             add=True)
            if with_cnt:
                pltpu.async_copy(ones, cntacc.at[sbuf.at[k]], cn, add=True)
        return carry
    lax.fori_loop(0, NGRP, _outer, 0)

    for p in range(2):
        pltpu.make_async_copy(frows.at[p], acc.at[sbuf.at[0]],
                              ssem[p]).wait()
    if with_cnt:
        def _dr(i, carry):
            pltpu.make_async_copy(ones, cntacc.at[sbuf.at[0]], cn).wait()
            return carry
        lax.fori_loop(0, CPS, _dr, 0)

    plsc.subcore_barrier()

    def _wb(j, carry):
        k = s + NS * j

        @pl.when(k < NWCH)
        def _():
            pltpu.sync_copy(acc.at[pl.ds(k * WCH, WCH)],
                            agg_out.at[c, pl.ds(k * WCH, WCH)])
            if with_cnt:
                pltpu.sync_copy(cntacc.at[pl.ds(k * WCH, WCH)],
                                cnt_out.at[c, pl.ds(k * WCH, WCH)])
        return carry
    lax.fori_loop(0, (NWCH + NS - 1) // NS, _wb, 0)


def _make_sc(with_cnt, nb, a):
    out_type = [jax.ShapeDtypeStruct((2, 2 * N, H), jnp.float32)]
    scratch = [
        pltpu.VMEM_SHARED((ACC_R, H), jnp.float32),
    ]
    if with_cnt:
        out_type.append(jax.ShapeDtypeStruct((2, 2 * N, 16), jnp.float32))
        scratch += [
            pltpu.VMEM_SHARED((ACC_R, 16), jnp.float32),
            pltpu.VMEM((CH, 16), jnp.float32),
            pltpu.VMEM((CH, 16), jnp.float32),
        ]
    scratch += [
        pltpu.VMEM((GRP, CH), jnp.int32),
        pltpu.VMEM((GRP, CH), jnp.int32),
        pltpu.VMEM((nb, CH, H // 2), jnp.int32),
        pltpu.VMEM((2, CH, H), jnp.float32),
    ]
    scratch += [pltpu.SemaphoreType.DMA] * (nb + 2)
    if with_cnt:
        scratch.append(pltpu.SemaphoreType.DMA)
    return pl.kernel(
        functools.partial(_sc_body, with_cnt, nb, a),
        out_type=tuple(out_type) if with_cnt else out_type[0],
        mesh=plsc.VectorSubcoreMesh(core_axis_name="c", subcore_axis_name="s"),
        scratch_types=scratch,
        compiler_params=pltpu.CompilerParams(use_tc_tiling_on_sc=False),
    )


_sc1 = _make_sc(True, 3, 2)
_sc2 = _make_sc(False, 6, 4)


def kernel(des, tweet, num_prop, cat_prop, edge_index, edge_type,
           W_cat, b_cat, W_in, b_in, W_rel, W_root, b_rgcn,
           W_o1, b_o1, W_o2, b_o2):
    src = edge_index[0].astype(jnp.int32)
    dst = edge_index[1].astype(jnp.int32)
    et = edge_type.astype(jnp.int32)
    g0 = jnp.pad(et * N + src, (0, EPAD - E))
    gx = jnp.stack([g0, g0 + 2 * N]).reshape(2, NCH, CH)
    sx = jnp.pad(et * N + dst, (0, EPAD - E),
                 constant_values=2 * N).reshape(NCH, CH)

    bc = b_cat.reshape(1, D)
    bi = b_in.reshape(1, D)
    br = b_rgcn.reshape(1, D)
    bo1 = b_o1.reshape(1, D)
    bo2 = b_o2.reshape(1, 2)

    def _shuf(t):
        tb = (t.reshape(4, N, 2, H // 2).transpose(0, 1, 3, 2)
              .reshape(4 * N, H // 2, 2).astype(jnp.bfloat16))
        return jax.lax.bitcast_convert_type(tb, jnp.int32)

    x1, t1 = _k1(cat_prop, W_cat, bc, W_in, bi, W_rel)
    agg1, cnt16 = _sc1(_shuf(t1), gx, sx)
    x2, t2 = _k2(x1, W_root, br, agg1, agg1, agg1, agg1, cnt16, cnt16, W_rel)
    agg2 = _sc2(_shuf(t2), gx, sx)
    return _k3(x2, W_root, br, agg2, agg2, agg2, agg2, cnt16, cnt16,
               W_o1, bo1, W_o2, bo2)


# speedup vs baseline: 1.1595x; 1.0737x over previous
import functools

import jax
import jax.numpy as jnp
from jax import lax
from jax.experimental import pallas as pl
from jax.experimental.pallas import tpu as pltpu
from jax.experimental.pallas import tpu_sc as plsc

N = 10000
E = 320000
D = 128
H = 64
Bn = 1000
NBLK = N // Bn

CH = 112
NS = 16
CPS = 180
NCH = CPS * NS
EPAD = NCH * CH
GRP = 12
NGRP = CPS // GRP
ACC_R = 2 * N + 8
NZF = 178
ZT = ACC_R - NZF * CH
WCH = 1000
NWCH = (2 * N) // WCH


def _leaky(v):
    return jnp.where(v >= 0, v, 0.01 * v)


def _table_write(x, wr_ref, t_ref):
    xr0 = jnp.dot(x, wr_ref[0])
    xr1 = jnp.dot(x, wr_ref[1])
    t_ref[0] = xr0[:, :H]
    t_ref[1] = xr1[:, :H]
    t_ref[2] = xr0[:, H:]
    t_ref[3] = xr1[:, H:]


def _k1_body(cat_ref, wc_ref, bc_ref, wi_ref, bi_ref, wr_ref, x_ref, t_ref):
    c = _leaky(jnp.dot(cat_ref[...], wc_ref[...]) + bc_ref[...])
    x = _leaky(jnp.dot(c, wi_ref[...]) + bi_ref[...])
    x_ref[...] = x
    _table_write(x, wr_ref, t_ref)


def _combine(x, wroot_ref, br_ref, a00, a01, a10, a11, c0, c1):
    cnt0 = jnp.maximum(c0[0][:, 0:1], 1.0)
    cnt1 = jnp.maximum(c1[0][:, 0:1], 1.0)
    lo = a00[0] / cnt0 + a01[0] / cnt1
    hi = a10[0] / cnt0 + a11[0] / cnt1
    return (jnp.dot(x, wroot_ref[...]) + br_ref[...]
            + jnp.concatenate([lo, hi], axis=1))


def _k2_body(x_ref, wroot_ref, br_ref, a00, a01, a10, a11, c0, c1, wr_ref,
             x2_ref, t_ref):
    x2 = _combine(x_ref[...], wroot_ref, br_ref, a00, a01, a10, a11, c0, c1)
    x2_ref[...] = x2
    _table_write(x2, wr_ref, t_ref)


def _k3_body(x_ref, wroot_ref, br_ref, a00, a01, a10, a11, c0, c1,
             wo1_ref, bo1_ref, wo2_ref, bo2_ref, out_ref):
    x3 = _combine(x_ref[...], wroot_ref, br_ref, a00, a01, a10, a11, c0, c1)
    x4 = _leaky(jnp.dot(x3, wo1_ref[...]) + bo1_ref[...])
    out_ref[...] = jnp.dot(x4, wo2_ref[...]) + bo2_ref[...]


def _full(shape):
    return pl.BlockSpec(shape, lambda i: (0,) * len(shape))


def _agg_specs():
    return [
        pl.BlockSpec((1, Bn, H), lambda i: (0, i, 0)),
        pl.BlockSpec((1, Bn, H), lambda i: (0, NBLK + i, 0)),
        pl.BlockSpec((1, Bn, H), lambda i: (1, i, 0)),
        pl.BlockSpec((1, Bn, H), lambda i: (1, NBLK + i, 0)),
    ]


def _cnt_specs():
    return [
        pl.BlockSpec((1, Bn, 16), lambda i: (0, i, 0)),
        pl.BlockSpec((1, Bn, 16), lambda i: (0, NBLK + i, 0)),
    ]


_k1 = pl.pallas_call(
    _k1_body,
    grid=(NBLK,),
    in_specs=[
        pl.BlockSpec((Bn, 11), lambda i: (i, 0)),
        _full((11, D)), _full((1, D)), _full((D, D)), _full((1, D)),
        _full((2, D, D)),
    ],
    out_specs=[
        pl.BlockSpec((Bn, D), lambda i: (i, 0)),
        pl.BlockSpec((4, Bn, H), lambda i: (0, i, 0)),
    ],
    out_shape=[
        jax.ShapeDtypeStruct((N, D), jnp.float32),
        jax.ShapeDtypeStruct((4, N, H), jnp.float32),
    ],
)

_k2 = pl.pallas_call(
    _k2_body,
    grid=(NBLK,),
    in_specs=[
        pl.BlockSpec((Bn, D), lambda i: (i, 0)),
        _full((D, D)), _full((1, D)),
        *_agg_specs(), *_cnt_specs(),
        _full((2, D, D)),
    ],
    out_specs=[
        pl.BlockSpec((Bn, D), lambda i: (i, 0)),
        pl.BlockSpec((4, Bn, H), lambda i: (0, i, 0)),
    ],
    out_shape=[
        jax.ShapeDtypeStruct((N, D), jnp.float32),
        jax.ShapeDtypeStruct((4, N, H), jnp.float32),
    ],
)

_k3 = pl.pallas_call(
    _k3_body,
    grid=(NBLK,),
    in_specs=[
        pl.BlockSpec((Bn, D), lambda i: (i, 0)),
        _full((D, D)), _full((1, D)),
        *_agg_specs(), *_cnt_specs(),
        _full((D, D)), _full((1, D)), _full((D, 2)), _full((1, 2)),
    ],
    out_specs=pl.BlockSpec((Bn, 2), lambda i: (i, 0)),
    out_shape=jax.ShapeDtypeStruct((N, 2), jnp.float32),
)


def _sc_body(with_cnt, nb, a, tab, gx, sx, *rest):
    if with_cnt:
        (agg_out, cnt_out, acc, cntacc, ones, zb16, gbuf, sbuf,
         rows) = rest[:9]
        gsem = list(rest[9:9 + nb])
        ssem = list(rest[9 + nb:9 + 2 * nb])
        cn = rest[9 + 2 * nb]
    else:
        (agg_out, acc, gbuf, sbuf, rows) = rest[:5]
        gsem = list(rest[5:5 + nb])
        ssem = list(rest[5 + nb:5 + 2 * nb])
        cnt_out = cntacc = ones = zb16 = cn = None
    c = lax.axis_index("c")
    s = lax.axis_index("s")

    def _zr(i, carry):
        def _zc(j, carry2):
            for b in range(nb):
                rows[b, i, pl.ds(j * 16, 16)] = jnp.zeros((16,), jnp.float32)
            return carry2
        return lax.fori_loop(0, H // 16, _zc, carry)
    lax.fori_loop(0, CH, _zr, 0)

    def _zs(i, carry):
        sbuf[0, pl.ds(i * 16, 16)] = jnp.zeros((16,), jnp.int32)
        return carry
    lax.fori_loop(0, CH // 16, _zs, 0)

    if with_cnt:
        def _zo(i, carry):
            zb16[i, pl.ds(0, 16)] = jnp.zeros((16,), jnp.float32)
            ones[i, pl.ds(0, 16)] = jnp.ones((16,), jnp.float32)
            return carry
        lax.fori_loop(0, CH, _zo, 0)

    def _za(j, carry):
        k = s + NS * j

        @pl.when(k < NZF)
        def _():
            pltpu.sync_copy(rows.at[0], acc.at[pl.ds(k * CH, CH)])
            if with_cnt:
                pltpu.sync_copy(zb16, cntacc.at[pl.ds(k * CH, CH)])

        @pl.when(k == NZF)
        def _():
            pltpu.sync_copy(rows.at[0, pl.ds(0, ZT)],
                            acc.at[pl.ds(NZF * CH, ZT)])
            if with_cnt:
                pltpu.sync_copy(zb16.at[pl.ds(0, ZT)],
                                cntacc.at[pl.ds(NZF * CH, ZT)])
        return carry
    lax.fori_loop(0, (NZF + NS) // NS + 1, _za, 0)

    plsc.subcore_barrier()

    for b in range(nb):
        pltpu.async_copy(rows.at[b], acc.at[sbuf.at[0]], ssem[b], add=True)

    def _outer(it, carry):
        base = s * CPS + it * GRP
        pltpu.sync_copy(gx.at[c, pl.ds(base, GRP)], gbuf)
        pltpu.sync_copy(sx.at[pl.ds(base, GRP)], sbuf)
        gd = [None] * GRP
        for k in range(a):
            b = k % nb
            pltpu.make_async_copy(rows.at[b], acc.at[sbuf.at[0]],
                                  ssem[b]).wait()
            gd[k] = pltpu.async_copy(tab.at[gbuf.at[k]], rows.at[b],
                                     gsem[b])
        for k in range(GRP):
            b = k % nb
            if k + a < GRP:
                b2 = (k + a) % nb
                pltpu.make_async_copy(rows.at[b2], acc.at[sbuf.at[0]],
                                      ssem[b2]).wait()
                gd[k + a] = pltpu.async_copy(tab.at[gbuf.at[k + a]],
                                             rows.at[b2], gsem[b2])
            gd[k].wait()
            pltpu.async_copy(rows.at[b], acc.at[sbuf.at[k]], ssem[b],
                             add=True)
            if with_cnt:
                pltpu.async_copy(ones, cntacc.at[sbuf.at[k]], cn, add=True)
        return carry
    lax.fori_loop(0, NGRP, _outer, 0)

    for b in range(nb):
        pltpu.make_async_copy(rows.at[b], acc.at[sbuf.at[0]], ssem[b]).wait()
    if with_cnt:
        def _dr(i, carry):
            pltpu.make_async_copy(ones, cntacc.at[sbuf.at[0]], cn).wait()
            return carry
        lax.fori_loop(0, CPS, _dr, 0)

    plsc.subcore_barrier()

    def _wb(j, carry):
        k = s + NS * j

        @pl.when(k < NWCH)
        def _():
            pltpu.sync_copy(acc.at[pl.ds(k * WCH, WCH)],
                            agg_out.at[c, pl.ds(k * WCH, WCH)])
            if with_cnt:
                pltpu.sync_copy(cntacc.at[pl.ds(k * WCH, WCH)],
                                cnt_out.at[c, pl.ds(k * WCH, WCH)])
        return carry
    lax.fori_loop(0, (NWCH + NS - 1) // NS, _wb, 0)


def _make_sc(with_cnt, nb, a):
    out_type = [jax.ShapeDtypeStruct((2, 2 * N, H), jnp.float32)]
    scratch = [
        pltpu.VMEM_SHARED((ACC_R, H), jnp.float32),
    ]
    if with_cnt:
        out_type.append(jax.ShapeDtypeStruct((2, 2 * N, 16), jnp.float32))
        scratch += [
            pltpu.VMEM_SHARED((ACC_R, 16), jnp.float32),
            pltpu.VMEM((CH, 16), jnp.float32),
            pltpu.VMEM((CH, 16), jnp.float32),
        ]
    scratch += [
        pltpu.VMEM((GRP, CH), jnp.int32),
        pltpu.VMEM((GRP, CH), jnp.int32),
        pltpu.VMEM((nb, CH, H), jnp.float32),
    ]
    scratch += [pltpu.SemaphoreType.DMA] * (2 * nb)
    if with_cnt:
        scratch.append(pltpu.SemaphoreType.DMA)
    return pl.kernel(
        functools.partial(_sc_body, with_cnt, nb, a),
        out_type=tuple(out_type) if with_cnt else out_type[0],
        mesh=plsc.VectorSubcoreMesh(core_axis_name="c", subcore_axis_name="s"),
        scratch_types=scratch,
        compiler_params=pltpu.CompilerParams(use_tc_tiling_on_sc=False),
    )


_sc1 = _make_sc(True, 3, 2)
_sc2 = _make_sc(False, 6, 4)


def kernel(des, tweet, num_prop, cat_prop, edge_index, edge_type,
           W_cat, b_cat, W_in, b_in, W_rel, W_root, b_rgcn,
           W_o1, b_o1, W_o2, b_o2):
    src = edge_index[0].astype(jnp.int32)
    dst = edge_index[1].astype(jnp.int32)
    et = edge_type.astype(jnp.int32)
    g0 = jnp.pad(et * N + src, (0, EPAD - E))
    gx = jnp.stack([g0, g0 + 2 * N]).reshape(2, NCH, CH)
    sx = jnp.pad(et * N + dst, (0, EPAD - E),
                 constant_values=2 * N).reshape(NCH, CH)

    bc = b_cat.reshape(1, D)
    bi = b_in.reshape(1, D)
    br = b_rgcn.reshape(1, D)
    bo1 = b_o1.reshape(1, D)
    bo2 = b_o2.reshape(1, 2)

    x1, t1 = _k1(cat_prop, W_cat, bc, W_in, bi, W_rel)
    agg1, cnt16 = _sc1(t1.reshape(4 * N, H), gx, sx)
    x2, t2 = _k2(x1, W_root, br, agg1, agg1, agg1, agg1, cnt16, cnt16, W_rel)
    agg2 = _sc2(t2.reshape(4 * N, H), gx, sx)
    return _k3(x2, W_root, br, agg2, agg2, agg2, agg2, cnt16, cnt16,
               W_o1, bo1, W_o2, bo2)
